# baseline (speedup 1.0000x reference)
import jax
import jax.numpy as jnp
from jax.experimental import pallas as pl
from jax.experimental.pallas import tpu as pltpu

_L = 786432
_G = 58


def _probe_body(s_ref, out_ref, acc_ref):
    g = pl.program_id(0)

    @pl.when(g == 0)
    def _init():
        acc_ref[0] = 0.0

    acc_ref[0] = acc_ref[0] + jnp.sum(s_ref[...])

    @pl.when(g == _G - 1)
    def _fin():
        out_ref[0] = acc_ref[0]


def kernel(predicted_locs, predicted_scores, true_locs, true_classes):
    flat = predicted_scores.reshape(-1)
    out = pl.pallas_call(
        _probe_body,
        grid=(_G,),
        in_specs=[pl.BlockSpec((_L,), lambda g: (g,))],
        out_specs=pl.BlockSpec(memory_space=pltpu.SMEM),
        out_shape=jax.ShapeDtypeStruct((1,), jnp.float32),
        scratch_shapes=[pltpu.SMEM((1,), jnp.float32)],
    )(flat)
    return out[0]


# speedup vs baseline: 4.8830x; 4.8830x over previous
import functools

import jax
import jax.numpy as jnp
from jax import lax
from jax.experimental import pallas as pl
from jax.experimental.pallas import tpu as pltpu
from jax.experimental.pallas import tpu_sc as plsc

_THRESHOLD = 0.5
_NEG_POS_RATIO = 3
_ALPHA = 1.0
_B, _N, _C = 64, 8732, 81
_N4 = _N * 4
_N2 = _N * 2
_FULL = _N // 16


def _loc_body(pd_hbm, td_hbm, cls_hbm, out_hbm, p_v, t_v, c_v, acc_v):
    wid = lax.axis_index("s") * 2 + lax.axis_index("c")
    lane = lax.iota(jnp.int32, 16)
    rep4 = lax.shift_right_logical(lane, 2)
    pltpu.sync_copy(cls_hbm.at[pl.ds(wid * _N2, _N2)], c_v)

    for rep in range(2):
        b = wid * 2 + rep
        pltpu.sync_copy(pd_hbm.at[b], p_v)
        pltpu.sync_copy(td_hbm.at[b], t_v)

        def half_step(i, a, rep=rep, ks=4, moff=0, ioff=0):
            m16 = c_v[pl.ds(rep * _N + i * 16 - moff, 16)]
            for k in range(ks):
                pv = p_v[pl.ds(i * 64 + k * 16, 16)]
                tv = t_v[pl.ds(i * 64 + k * 16, 16)]
                cv = jnp.take(m16, ioff + 4 * k + rep4)
                ad = jnp.abs(pv - tv)
                s1 = jnp.where(ad < 1.0, 0.5 * ad * ad, ad - 0.5)
                a = a + jnp.where(cv != 0, s1, 0.0)
            return a

        acc = lax.fori_loop(0, _FULL, half_step, jnp.zeros((16,), jnp.float32))
        acc = half_step(_FULL, acc, rep=rep, ks=3, moff=4, ioff=4)
        acc_v[...] = acc
        pltpu.sync_copy(acc_v, out_hbm.at[b])


def _loc_call():
    return functools.partial(
        pl.kernel,
        mesh=plsc.VectorSubcoreMesh(core_axis_name="c", subcore_axis_name="s"),
        out_type=jax.ShapeDtypeStruct((_B, 16), jnp.float32),
        scratch_types=[
            pltpu.VMEM((_N4,), jnp.float32),
            pltpu.VMEM((_N4,), jnp.float32),
            pltpu.VMEM((_N2,), jnp.int32),
            pltpu.VMEM((16,), jnp.float32),
        ],
    )(_loc_body)


def _conf_body(cls_ref, s_ref, loc_ref, out_ref, acc_ref):
    b = pl.program_id(0)

    @pl.when(b == 0)
    def _init():
        acc_ref[0] = 0.0
        acc_ref[1] = 0.0
        acc_ref[2] = 0.0

    s = s_ref[0].astype(jnp.bfloat16)
    e = jnp.exp(s)
    lse = jnp.log(jnp.sum(e, axis=1).astype(jnp.float32))

    cls = cls_ref[0, 0]
    clsb = cls.astype(jnp.int16)
    col = jax.lax.broadcasted_iota(jnp.int16, (_N, _C), 1)
    s_true = jnp.sum(
        jnp.where(col == clsb[:, None], s, jnp.bfloat16(0.0)), axis=1
    ).astype(jnp.float32)
    ce = lse - s_true

    pos = cls != 0
    posf = pos.astype(jnp.float32)
    npos = jnp.sum(posf)
    conf_pos = jnp.sum(ce * posf)
    ce_neg = jnp.where(pos, 0.0, ce)
    sum_neg = jnp.sum(ce_neg)

    acc_ref[0] = acc_ref[0] + conf_pos
    acc_ref[2] = acc_ref[2] + npos

    k_f = jnp.float32(_NEG_POS_RATIO) * npos
    n_neg = jnp.float32(_N) - npos
    fast = k_f >= n_neg

    @pl.when(fast)
    def _all_negatives():
        acc_ref[1] = acc_ref[1] + sum_neg

    @pl.when(jnp.logical_not(fast))
    def _topk():
        def step(i, rb):
            cand = rb | (jnp.int32(1) << (30 - i))
            t = jax.lax.bitcast_convert_type(cand, jnp.float32)
            cnt = jnp.sum(jnp.where(ce_neg >= t, 1.0, 0.0))
            return jnp.where(cnt >= k_f, cand, rb)

        rb = jax.lax.fori_loop(0, 31, step, jnp.int32(0))
        t = jax.lax.bitcast_convert_type(rb, jnp.float32)
        gt = ce_neg > t
        cnt_gt = jnp.sum(gt.astype(jnp.float32))
        sum_gt = jnp.sum(jnp.where(gt, ce_neg, 0.0))
        acc_ref[1] = acc_ref[1] + jnp.where(
            k_f > 0.0, sum_gt + (k_f - cnt_gt) * t, 0.0
        )

    @pl.when(b == _B - 1)
    def _finish():
        denom = jnp.maximum(acc_ref[2], 1.0)
        loc = jnp.sum(loc_ref[...])
        out_ref[0] = (acc_ref[0] + acc_ref[1]) / denom + _ALPHA * loc / denom


def _conf_call(interpret=False):
    return pl.pallas_call(
        _conf_body,
        grid=(_B,),
        in_specs=[
            pl.BlockSpec((1, 1, _N), lambda b: (b, 0, 0)),
            pl.BlockSpec((1, _N, _C), lambda b: (b, 0, 0)),
            pl.BlockSpec((_B, 16), lambda b: (0, 0)),
        ],
        out_specs=pl.BlockSpec(memory_space=pltpu.SMEM),
        out_shape=jax.ShapeDtypeStruct((1,), jnp.float32),
        scratch_shapes=[pltpu.SMEM((3,), jnp.float32)],
        interpret=interpret,
    )


def kernel(predicted_locs, predicted_scores, true_locs, true_classes):
    cls3 = true_classes.reshape(_B, 1, _N)
    pd_f = predicted_locs.reshape(_B, _N4)
    td_f = true_locs.reshape(_B, _N4)
    cls_flat = true_classes.reshape(_B * _N)
    loc_part = _loc_call()(pd_f, td_f, cls_flat)
    out = _conf_call()(cls3, predicted_scores, loc_part)
    return out[0]


# speedup vs baseline: 4.8834x; 1.0001x over previous
import functools

import jax
import jax.numpy as jnp
from jax import lax
from jax.experimental import pallas as pl
from jax.experimental.pallas import tpu as pltpu
from jax.experimental.pallas import tpu_sc as plsc

_THRESHOLD = 0.5
_NEG_POS_RATIO = 3
_ALPHA = 1.0
_B, _N, _C = 64, 8732, 81
_N4 = _N * 4
_N2 = _N * 2
_FULL = _N // 16


def _loc_body(pd_hbm, td_hbm, cls_hbm, out_hbm, p_v, t_v, c_v, acc_v):
    wid = lax.axis_index("s") * 2 + lax.axis_index("c")
    lane = lax.iota(jnp.int32, 16)
    rep4 = lax.shift_right_logical(lane, 2)
    pltpu.sync_copy(cls_hbm.at[pl.ds(wid * _N2, _N2)], c_v)

    for rep in range(2):
        b = wid * 2 + rep
        pltpu.sync_copy(pd_hbm.at[b], p_v)
        pltpu.sync_copy(td_hbm.at[b], t_v)

        def half_step(i, a, rep=rep, ks=4, moff=0, ioff=0):
            m16 = c_v[pl.ds(rep * _N + i * 16 - moff, 16)]
            for k in range(ks):
                pv = p_v[pl.ds(i * 64 + k * 16, 16)]
                tv = t_v[pl.ds(i * 64 + k * 16, 16)]
                cv = jnp.take(m16, ioff + 4 * k + rep4)
                ad = jnp.abs(pv - tv)
                s1 = jnp.where(ad < 1.0, 0.5 * ad * ad, ad - 0.5)
                a = a + jnp.where(cv != 0, s1, 0.0)
            return a

        acc = lax.fori_loop(0, _FULL, half_step, jnp.zeros((16,), jnp.float32))
        acc = half_step(_FULL, acc, rep=rep, ks=3, moff=4, ioff=4)
        acc_v[...] = acc
        pltpu.sync_copy(acc_v, out_hbm.at[b])


def _loc_call():
    return functools.partial(
        pl.kernel,
        mesh=plsc.VectorSubcoreMesh(core_axis_name="c", subcore_axis_name="s"),
        out_type=jax.ShapeDtypeStruct((_B, 16), jnp.float32),
        scratch_types=[
            pltpu.VMEM((_N4,), jnp.float32),
            pltpu.VMEM((_N4,), jnp.float32),
            pltpu.VMEM((_N2,), jnp.int32),
            pltpu.VMEM((16,), jnp.float32),
        ],
    )(_loc_body)


def _conf_body(cls_ref, s_ref, loc_ref, out_ref, acc_ref):
    b = pl.program_id(0)

    @pl.when(b == 0)
    def _init():
        acc_ref[0] = 0.0
        acc_ref[1] = 0.0
        acc_ref[2] = 0.0

    s = s_ref[0].astype(jnp.bfloat16)
    e = jnp.exp(s)
    lse = jnp.log(jnp.sum(e, axis=1).astype(jnp.float32))

    cls = cls_ref[0, 0]
    clsb = cls.astype(jnp.int16)
    col = jax.lax.broadcasted_iota(jnp.int16, (_N, _C), 1)
    s_true = jnp.sum(
        jnp.where(col == clsb[:, None], s, jnp.bfloat16(0.0)), axis=1
    ).astype(jnp.float32)
    ce = lse - s_true

    pos = cls != 0
    posf = pos.astype(jnp.float32)
    npos = jnp.sum(posf)
    conf_pos = jnp.sum(ce * posf)
    ce_neg = jnp.where(pos, 0.0, ce)
    sum_neg = jnp.sum(ce_neg)

    acc_ref[0] = acc_ref[0] + conf_pos
    acc_ref[2] = acc_ref[2] + npos

    k_f = jnp.float32(_NEG_POS_RATIO) * npos
    n_neg = jnp.float32(_N) - npos
    fast = k_f >= n_neg

    @pl.when(fast)
    def _all_negatives():
        acc_ref[1] = acc_ref[1] + sum_neg

    @pl.when(jnp.logical_not(fast))
    def _topk():
        def step(i, rb):
            cand = rb | (jnp.int32(1) << (30 - i))
            t = jax.lax.bitcast_convert_type(cand, jnp.float32)
            cnt = jnp.sum(jnp.where(ce_neg >= t, 1.0, 0.0))
            return jnp.where(cnt >= k_f, cand, rb)

        rb = jax.lax.fori_loop(0, 31, step, jnp.int32(0))
        t = jax.lax.bitcast_convert_type(rb, jnp.float32)
        gt = ce_neg > t
        cnt_gt = jnp.sum(gt.astype(jnp.float32))
        sum_gt = jnp.sum(jnp.where(gt, ce_neg, 0.0))
        acc_ref[1] = acc_ref[1] + jnp.where(
            k_f > 0.0, sum_gt + (k_f - cnt_gt) * t, 0.0
        )

    @pl.when(b == _B - 1)
    def _finish():
        denom = jnp.maximum(acc_ref[2], 1.0)
        loc = jnp.sum(loc_ref[...])
        out_ref[0] = (acc_ref[0] + acc_ref[1]) / denom + _ALPHA * loc / denom


def _conf_call():
    return pl.pallas_call(
        _conf_body,
        grid=(_B,),
        in_specs=[
            pl.BlockSpec((1, 1, _N), lambda b: (b, 0, 0)),
            pl.BlockSpec((1, _N, _C), lambda b: (b, 0, 0)),
            pl.BlockSpec((_B, 16), lambda b: (0, 0)),
        ],
        out_specs=pl.BlockSpec(memory_space=pltpu.SMEM),
        out_shape=jax.ShapeDtypeStruct((1,), jnp.float32),
        scratch_shapes=[pltpu.SMEM((3,), jnp.float32)],
    )


def kernel(predicted_locs, predicted_scores, true_locs, true_classes):
    cls3 = true_classes.reshape(_B, 1, _N)
    pd_f = predicted_locs.reshape(_B, _N4)
    td_f = true_locs.reshape(_B, _N4)
    cls_flat = true_classes.reshape(_B * _N)
    loc_part = _loc_call()(pd_f, td_f, cls_flat)
    out = _conf_call()(cls3, predicted_scores, loc_part)
    return out[0]


# speedup vs baseline: 4.8857x; 1.0005x over previous
import functools

import jax
import jax.numpy as jnp
from jax import lax
from jax.experimental import pallas as pl
from jax.experimental.pallas import tpu as pltpu
from jax.experimental.pallas import tpu_sc as plsc

_THRESHOLD = 0.5
_NEG_POS_RATIO = 3
_ALPHA = 1.0
_B, _N, _C = 64, 8732, 81
_N4 = _N * 4
_N2 = _N * 2
_FULL = _N // 16


def _loc_body(pd_hbm, td_hbm, cls_hbm, out_hbm, p_v, t_v, c_v, acc_v):
    wid = lax.axis_index("s") * 2 + lax.axis_index("c")
    lane = lax.iota(jnp.int32, 16)
    rep4 = lax.shift_right_logical(lane, 2)
    pltpu.sync_copy(cls_hbm.at[pl.ds(wid * _N2, _N2)], c_v)

    for rep in range(2):
        b = wid * 2 + rep
        pltpu.sync_copy(pd_hbm.at[b], p_v)
        pltpu.sync_copy(td_hbm.at[b], t_v)

        def half_step(i, a, rep=rep, ks=4, moff=0, ioff=0):
            m16 = c_v[pl.ds(rep * _N + i * 16 - moff, 16)]
            for k in range(ks):
                pv = p_v[pl.ds(i * 64 + k * 16, 16)]
                tv = t_v[pl.ds(i * 64 + k * 16, 16)]
                cv = jnp.take(m16, ioff + 4 * k + rep4)
                ad = jnp.abs(pv - tv)
                s1 = jnp.where(ad < 1.0, 0.5 * ad * ad, ad - 0.5)
                a = a + jnp.where(cv != 0, s1, 0.0)
            return a

        acc = lax.fori_loop(0, _FULL, half_step, jnp.zeros((16,), jnp.float32))
        acc = half_step(_FULL, acc, rep=rep, ks=3, moff=4, ioff=4)
        acc_v[...] = acc
        pltpu.sync_copy(acc_v, out_hbm.at[b])


def _loc_call():
    return functools.partial(
        pl.kernel,
        mesh=plsc.VectorSubcoreMesh(core_axis_name="c", subcore_axis_name="s"),
        compiler_params=pltpu.CompilerParams(use_tc_tiling_on_sc=True),
        out_type=jax.ShapeDtypeStruct((_B, 16), jnp.float32),
        scratch_types=[
            pltpu.VMEM((_N4,), jnp.float32),
            pltpu.VMEM((_N4,), jnp.float32),
            pltpu.VMEM((_N2,), jnp.int32),
            pltpu.VMEM((16,), jnp.float32),
        ],
    )(_loc_body)


def _conf_body(cls_ref, s_ref, loc_ref, out_ref, acc_ref):
    b = pl.program_id(0)

    @pl.when(b == 0)
    def _init():
        acc_ref[0] = 0.0
        acc_ref[1] = 0.0
        acc_ref[2] = 0.0

    s = s_ref[0].astype(jnp.bfloat16)
    e = jnp.exp(s)
    lse = jnp.log(jnp.sum(e, axis=1).astype(jnp.float32))

    cls = cls_ref[0, 0]
    clsb = cls.astype(jnp.int16)
    col = jax.lax.broadcasted_iota(jnp.int16, (_N, _C), 1)
    s_true = jnp.sum(
        jnp.where(col == clsb[:, None], s, jnp.bfloat16(0.0)), axis=1
    ).astype(jnp.float32)
    ce = lse - s_true

    pos = cls != 0
    posf = pos.astype(jnp.float32)
    npos = jnp.sum(posf)
    conf_pos = jnp.sum(ce * posf)
    ce_neg = jnp.where(pos, 0.0, ce)
    sum_neg = jnp.sum(ce_neg)

    acc_ref[0] = acc_ref[0] + conf_pos
    acc_ref[2] = acc_ref[2] + npos

    k_f = jnp.float32(_NEG_POS_RATIO) * npos
    n_neg = jnp.float32(_N) - npos
    fast = k_f >= n_neg

    @pl.when(fast)
    def _all_negatives():
        acc_ref[1] = acc_ref[1] + sum_neg

    @pl.when(jnp.logical_not(fast))
    def _topk():
        def step(i, rb):
            cand = rb | (jnp.int32(1) << (30 - i))
            t = jax.lax.bitcast_convert_type(cand, jnp.float32)
            cnt = jnp.sum(jnp.where(ce_neg >= t, 1.0, 0.0))
            return jnp.where(cnt >= k_f, cand, rb)

        rb = jax.lax.fori_loop(0, 31, step, jnp.int32(0))
        t = jax.lax.bitcast_convert_type(rb, jnp.float32)
        gt = ce_neg > t
        cnt_gt = jnp.sum(gt.astype(jnp.float32))
        sum_gt = jnp.sum(jnp.where(gt, ce_neg, 0.0))
        acc_ref[1] = acc_ref[1] + jnp.where(
            k_f > 0.0, sum_gt + (k_f - cnt_gt) * t, 0.0
        )

    @pl.when(b == _B - 1)
    def _finish():
        denom = jnp.maximum(acc_ref[2], 1.0)
        loc = jnp.sum(loc_ref[...])
        out_ref[0] = (acc_ref[0] + acc_ref[1]) / denom + _ALPHA * loc / denom


def _conf_call():
    return pl.pallas_call(
        _conf_body,
        grid=(_B,),
        in_specs=[
            pl.BlockSpec((1, 1, _N), lambda b: (b, 0, 0)),
            pl.BlockSpec((1, _N, _C), lambda b: (b, 0, 0)),
            pl.BlockSpec((_B, 16), lambda b: (0, 0)),
        ],
        out_specs=pl.BlockSpec(memory_space=pltpu.SMEM),
        out_shape=jax.ShapeDtypeStruct((1,), jnp.float32),
        scratch_shapes=[pltpu.SMEM((3,), jnp.float32)],
    )


def kernel(predicted_locs, predicted_scores, true_locs, true_classes):
    cls3 = true_classes.reshape(_B, 1, _N)
    pd_f = predicted_locs.reshape(_B, _N4)
    td_f = true_locs.reshape(_B, _N4)
    cls_flat = true_classes.reshape(_B * _N)
    loc_part = _loc_call()(pd_f, td_f, cls_flat)
    out = _conf_call()(cls3, predicted_scores, loc_part)
    return out[0]
